# baseline (speedup 1.0000x reference)
import jax
import jax.numpy as jnp
import numpy as np
from jax import lax
from jax.experimental import pallas as pl
from jax.experimental.pallas import tpu as pltpu
from jax.experimental.pallas import tpu_sc as plsc

_H = 0.1
_DT = 1.0 / 60
_MAX_VEL = 0.5 * 0.1 / _DT
_REST = 17510.1
_VISC = 60.0
_EPS = 1e-6
_N = 100000
_E = 6400000
_CSPIKY = 15.0 / (np.pi * _H ** 6)
_CDSPIKY = -45.0 / (np.pi * _H ** 6)
_W0 = _CSPIKY * _H ** 3

_NC = 2
_NS = 16
_L = 16
_NW = _NC * _NS
_SUB = 128
_NSUB = 16
_K = _SUB * _NSUB
_NCHUNKS = _E // _K
_CPW = -(-_NCHUNKS // _NW)
_NP = 100096
_ROWS = _NP // _NS


def _rsqrt(s2):
    i = lax.bitcast_convert_type(s2, jnp.int32)
    i = jnp.int32(0x5F3759DF) - lax.shift_right_arithmetic(i, 1)
    y = lax.bitcast_convert_type(i, jnp.float32)
    for _ in range(3):
        y = y * (1.5 - 0.5 * s2 * y * y)
    return y


def _make_pass(mode):
    TW = {"density": 3, "dp": 4, "xsph": 6}[mode]
    CW = {"density": 5, "dp": 3, "xsph": 3}[mode]

    def body(*refs):
        tabs_hbm = refs[:TW]
        src_hbm, dst_hbm, zeros_hbm = refs[TW:TW + 3]
        outs_hbm = refs[TW + 3:TW + 3 + CW]
        sc = refs[TW + 3 + CW:]
        accs_sp = sc[:CW]
        src_i, dst_i = sc[CW:CW + 2]
        gs = sc[CW + 2:CW + 2 + TW]
        gd = sc[CW + 2 + TW:CW + 2 + 2 * TW]
        contrib = sc[CW + 2 + 2 * TW:CW + 2 + 2 * TW + CW]
        bounce = sc[-2]
        sem = sc[-1]

        c = lax.axis_index("c")
        s = lax.axis_index("s")
        wid = c * _NS + s

        pltpu.sync_copy(zeros_hbm, bounce)
        for a_sp in accs_sp:
            pltpu.sync_copy(bounce, a_sp.at[pl.ds(s * _ROWS, _ROWS)])

        plsc.subcore_barrier()

        def compute_row(j):
            def vreg(i, _):
                def sl(ref):
                    return ref[j, pl.ds(i * _L, _L)]

                ddx = sl(gd[0]) - sl(gs[0])
                ddy = sl(gd[1]) - sl(gs[1])
                ddz = sl(gd[2]) - sl(gs[2])
                s2 = ddx * ddx + ddy * ddy + ddz * ddz + 1e-12
                y = _rsqrt(s2)
                r = s2 * y
                hmr = _H - r
                lt = r < _H
                hmr2 = hmr * hmr

                def st(k, val):
                    contrib[k][j, pl.ds(i * _L, _L)] = val

                if mode == "density":
                    w = jnp.where(lt, _CSPIKY * hmr2 * hmr, 0.0)
                    coef = jnp.where(lt, (_CDSPIKY / _REST) * hmr2 * y, 0.0)
                    st(0, w)
                    st(1, coef * ddx)
                    st(2, coef * ddy)
                    st(3, coef * ddz)
                    st(4, coef * coef * (s2 - 1e-12))
                elif mode == "dp":
                    coef = jnp.where(lt, (_CDSPIKY / _REST) * hmr2 * y, 0.0)
                    f = (sl(gs[3]) + sl(gd[3])) * coef
                    st(0, f * ddx)
                    st(1, f * ddy)
                    st(2, f * ddz)
                else:
                    w = jnp.where(lt, _CSPIKY * hmr2 * hmr, 0.0)
                    st(0, (sl(gs[3]) - sl(gd[3])) * w)
                    st(1, (sl(gs[4]) - sl(gd[4])) * w)
                    st(2, (sl(gs[5]) - sl(gd[5])) * w)
                return 0

            lax.fori_loop(0, _SUB // _L, vreg, 0)

        def chunk(g, _):
            cidx = wid + g * _NW

            @pl.when(cidx < _NCHUNKS)
            def _():
                rbase = cidx * _NSUB
                pltpu.sync_copy(src_hbm.at[pl.ds(rbase, _NSUB)], src_i)
                pltpu.sync_copy(dst_hbm.at[pl.ds(rbase, _NSUB)], dst_i)
                cps = []
                for j in range(_NSUB):
                    for t in range(TW):
                        cps.append(pltpu.async_copy(
                            tabs_hbm[t].at[src_i.at[j]], gs[t].at[j], sem))
                        cps.append(pltpu.async_copy(
                            tabs_hbm[t].at[dst_i.at[j]], gd[t].at[j], sem))
                for cp in cps:
                    cp.wait()
                for j in range(_NSUB):
                    compute_row(j)
                cps = []
                for j in range(_NSUB):
                    for k in range(CW):
                        cps.append(pltpu.async_copy(
                            contrib[k].at[j], accs_sp[k].at[dst_i.at[j]],
                            sem, add=True))
                for cp in cps:
                    cp.wait()

            return 0

        lax.fori_loop(0, _CPW, chunk, 0)

        plsc.subcore_barrier()
        for a_sp, o_hbm in zip(accs_sp, outs_hbm):
            pltpu.sync_copy(a_sp.at[pl.ds(s * _ROWS, _ROWS)], bounce)
            pltpu.sync_copy(bounce,
                            o_hbm.at[pl.ds(c * _NP + s * _ROWS, _ROWS)])

    return pl.kernel(
        body,
        out_type=tuple(jax.ShapeDtypeStruct((_NC * _NP,), jnp.float32)
                       for _ in range(CW)),
        mesh=plsc.VectorSubcoreMesh(core_axis_name="c", subcore_axis_name="s"),
        scratch_types=(
            [pltpu.VMEM_SHARED((_NP,), jnp.float32) for _ in range(CW)] +
            [pltpu.VMEM((_NSUB, _SUB), jnp.int32) for _ in range(2)] +
            [pltpu.VMEM((_NSUB, _SUB), jnp.float32) for _ in range(2 * TW)] +
            [pltpu.VMEM((_NSUB, _SUB), jnp.float32) for _ in range(CW)] +
            [pltpu.VMEM((_ROWS,), jnp.float32)] +
            [pltpu.SemaphoreType.DMA]
        ),
        name=f"fluid_{mode}",
    )


_pass_density = _make_pass("density")
_pass_dp = _make_pass("dp")
_pass_xsph = _make_pass("xsph")


def _pad(x):
    return jnp.concatenate([x, jnp.zeros((_NP - _N,), jnp.float32)])


def _combine(o):
    h = o.reshape(_NC, _NP)
    return h[0, :_N] + h[1, :_N]


def kernel(locs, vel, edge_index):
    src2 = edge_index[0].reshape(_E // _SUB, _SUB)
    dst2 = edge_index[1].reshape(_E // _SUB, _SUB)
    zrows = jnp.zeros((_ROWS,), jnp.float32)

    gravity = jnp.array([0.0, -9.8, 0.0], jnp.float32)
    v = vel + _DT * gravity
    speed = jnp.sqrt(jnp.sum(v * v, axis=-1, keepdims=True) + 1e-12)
    v = v * jnp.minimum(1.0, _MAX_VEL / speed)
    pos = locs + _DT * v

    for _ in range(3):
        px, py, pz = (_pad(pos[:, k]) for k in range(3))
        ow, ogx, ogy, ogz, og2 = _pass_density(px, py, pz, src2, dst2, zrows)
        rho = _combine(ow) + _W0
        C = rho / _REST - 1.0
        gx, gy, gz = _combine(ogx), _combine(ogy), _combine(ogz)
        sum_g2 = _combine(og2) + gx * gx + gy * gy + gz * gz
        lam = -C / (sum_g2 + _EPS)
        odx, ody, odz = _pass_dp(px, py, pz, _pad(lam), src2, dst2, zrows)
        dp = jnp.stack([_combine(odx), _combine(ody), _combine(odz)], axis=1)
        pos = pos + dp

    new_vel = (pos - locs) / _DT
    px, py, pz = (_pad(pos[:, k]) for k in range(3))
    vx, vy, vz = (_pad(new_vel[:, k]) for k in range(3))
    oxx, oxy, oxz = _pass_xsph(px, py, pz, vx, vy, vz, src2, dst2, zrows)
    xsph = jnp.stack([_combine(oxx), _combine(oxy), _combine(oxz)], axis=1)
    new_vel = new_vel + (0.01 * _VISC * _DT / _REST) * xsph
    return jnp.stack([pos, new_vel])


# speedup vs baseline: 2.3578x; 2.3578x over previous
import jax
import jax.numpy as jnp
import numpy as np
from jax import lax
from jax.experimental import pallas as pl
from jax.experimental.pallas import tpu as pltpu
from jax.experimental.pallas import tpu_sc as plsc

_H = 0.1
_DT = 1.0 / 60
_MAX_VEL = 0.5 * 0.1 / _DT
_REST = 17510.1
_VISC = 60.0
_EPS = 1e-6
_N = 100000
_E = 6400000
_CSPIKY = 15.0 / (np.pi * _H ** 6)
_CDSPIKY = -45.0 / (np.pi * _H ** 6)
_W0 = _CSPIKY * _H ** 3

_NC = 2
_NS = 16
_L = 16
_NW = _NC * _NS
_SUB = 128
_NSUB = 16
_K = _SUB * _NSUB
_NCHUNKS = _E // _K
_CPW = -(-_NCHUNKS // _NW)
_NP = 100096
_ROWS = _NP // _NS


def _rsqrt(s2):
    i = lax.bitcast_convert_type(s2, jnp.int32)
    i = jnp.int32(0x5F3759DF) - lax.shift_right_arithmetic(i, 1)
    y = lax.bitcast_convert_type(i, jnp.float32)
    for _ in range(3):
        y = y * (1.5 - 0.5 * s2 * y * y)
    return y


def _make_pass(mode):
    TW = {"density": 3, "dp": 1, "xsph": 6}[mode]
    CW = {"density": 5, "dp": 3, "xsph": 3}[mode]
    SP = TW
    EC = 4 if mode in ("density", "dp") else 0

    def body(*refs):
        it = iter(refs)
        tabs_hbm = [next(it) for _ in range(TW)]
        src_hbm, dst_hbm, zeros_hbm = (next(it) for _ in range(3))
        cache_hbm = [next(it) for _ in range(EC)]
        outs_hbm = [next(it) for _ in range(CW)]
        tabs_sp = [next(it) for _ in range(SP)]
        accs_sp = [next(it) for _ in range(CW)]
        src_i = next(it)
        dst_i = next(it)
        gs = [next(it) for _ in range(TW)]
        gd = [next(it) for _ in range(TW)]
        contrib = [next(it) for _ in range(CW)]
        cache_v = [next(it) for _ in range(EC)]
        bounce = next(it)
        sem = next(it)
        lsem = next(it)

        c = lax.axis_index("c")
        s = lax.axis_index("s")
        wid = c * _NS + s

        for t_hbm, t_sp in zip(tabs_hbm[:SP], tabs_sp):
            pltpu.sync_copy(t_hbm.at[pl.ds(s * _ROWS, _ROWS)], bounce)
            pltpu.sync_copy(bounce, t_sp.at[pl.ds(s * _ROWS, _ROWS)])
        pltpu.sync_copy(zeros_hbm, bounce)
        for a_sp in accs_sp:
            pltpu.sync_copy(bounce, a_sp.at[pl.ds(s * _ROWS, _ROWS)])

        plsc.subcore_barrier()

        def compute(i, j):
            def sl(ref):
                return ref[j, pl.ds(i * _L, _L)]

            def st(k, val):
                contrib[k][j, pl.ds(i * _L, _L)] = val

            if mode == "dp":
                coef = sl(cache_v[0])
                f = (sl(gs[0]) + sl(gd[0])) * coef
                st(0, f * sl(cache_v[1]))
                st(1, f * sl(cache_v[2]))
                st(2, f * sl(cache_v[3]))
                return

            ddx = sl(gd[0]) - sl(gs[0])
            ddy = sl(gd[1]) - sl(gs[1])
            ddz = sl(gd[2]) - sl(gs[2])
            s2 = ddx * ddx + ddy * ddy + ddz * ddz + 1e-12
            y = _rsqrt(s2)
            r = s2 * y
            hmr = _H - r
            lt = r < _H
            hmr2 = hmr * hmr
            if mode == "density":
                w = jnp.where(lt, _CSPIKY * hmr2 * hmr, 0.0)
                coef = jnp.where(lt, (_CDSPIKY / _REST) * hmr2 * y, 0.0)
                st(0, w)
                st(1, coef * ddx)
                st(2, coef * ddy)
                st(3, coef * ddz)
                st(4, coef * coef * (s2 - 1e-12))
                cache_v[0][j, pl.ds(i * _L, _L)] = coef
                cache_v[1][j, pl.ds(i * _L, _L)] = ddx
                cache_v[2][j, pl.ds(i * _L, _L)] = ddy
                cache_v[3][j, pl.ds(i * _L, _L)] = ddz
            else:
                w = jnp.where(lt, _CSPIKY * hmr2 * hmr, 0.0)
                st(0, (sl(gs[3]) - sl(gd[3])) * w)
                st(1, (sl(gs[4]) - sl(gd[4])) * w)
                st(2, (sl(gs[5]) - sl(gd[5])) * w)

        def chunk(g, _):
            cidx = wid + g * _NW

            @pl.when(cidx < _NCHUNKS)
            def _():
                rbase = cidx * _NSUB
                pltpu.sync_copy(src_hbm.at[pl.ds(rbase, _NSUB)], src_i)
                pltpu.sync_copy(dst_hbm.at[pl.ds(rbase, _NSUB)], dst_i)
                cps = []
                for j in range(_NSUB):
                    for t in range(TW):
                        tab = tabs_sp[t] if t < SP else tabs_hbm[t]
                        cps.append(pltpu.async_copy(
                            tab.at[src_i.at[j]], gs[t].at[j], sem))
                        cps.append(pltpu.async_copy(
                            tab.at[dst_i.at[j]], gd[t].at[j], sem))
                if mode == "dp":
                    for k in range(EC):
                        cps.append(pltpu.async_copy(
                            cache_hbm[k].at[pl.ds(rbase, _NSUB)],
                            cache_v[k], lsem))
                for cp in cps:
                    cp.wait()

                for j in range(_NSUB):
                    lax.fori_loop(
                        0, _SUB // _L,
                        lambda i, _, j=j: (compute(i, j), 0)[1], 0)

                cps = []
                if mode == "density":
                    for k in range(EC):
                        cps.append(pltpu.async_copy(
                            cache_v[k],
                            cache_hbm[k].at[pl.ds(rbase, _NSUB)], lsem))
                for j in range(_NSUB):
                    for k in range(CW):
                        cps.append(pltpu.async_copy(
                            contrib[k].at[j], accs_sp[k].at[dst_i.at[j]],
                            sem, add=True))
                for cp in cps:
                    cp.wait()

            return 0

        lax.fori_loop(0, _CPW, chunk, 0)

        plsc.subcore_barrier()
        for a_sp, o_hbm in zip(accs_sp, outs_hbm):
            pltpu.sync_copy(a_sp.at[pl.ds(s * _ROWS, _ROWS)], bounce)
            pltpu.sync_copy(bounce,
                            o_hbm.at[pl.ds(c * _NP + s * _ROWS, _ROWS)])

    out_type = (
        tuple(jax.ShapeDtypeStruct((_NC * _NP,), jnp.float32)
              for _ in range(CW)) +
        tuple(jax.ShapeDtypeStruct((_E // _SUB, _SUB), jnp.float32)
              for _ in range(EC if mode == "density" else 0)))

    def reorder_body(*refs):
        if mode != "density":
            return body(*refs)
        tabs = refs[:TW]
        rest = refs[TW:TW + 3]
        outs = refs[TW + 3:TW + 3 + CW]
        cache = refs[TW + 3 + CW:TW + 3 + CW + EC]
        sc = refs[TW + 3 + CW + EC:]
        return body(*tabs, *rest, *cache, *outs, *sc)

    n_in_cache = EC if mode == "dp" else 0
    return pl.kernel(
        reorder_body,
        out_type=out_type,
        mesh=plsc.VectorSubcoreMesh(core_axis_name="c", subcore_axis_name="s"),
        scratch_types=(
            [pltpu.VMEM_SHARED((_NP,), jnp.float32) for _ in range(SP)] +
            [pltpu.VMEM_SHARED((_NP,), jnp.float32) for _ in range(CW)] +
            [pltpu.VMEM((_NSUB, _SUB), jnp.int32) for _ in range(2)] +
            [pltpu.VMEM((_NSUB, _SUB), jnp.float32) for _ in range(2 * TW)] +
            [pltpu.VMEM((_NSUB, _SUB), jnp.float32) for _ in range(CW)] +
            [pltpu.VMEM((_NSUB, _SUB), jnp.float32) for _ in range(EC)] +
            [pltpu.VMEM((_ROWS,), jnp.float32)] +
            [pltpu.SemaphoreType.DMA, pltpu.SemaphoreType.DMA]
        ),
        name=f"fluid_{mode}",
    )


_pass_density = _make_pass("density")
_pass_dp = _make_pass("dp")
_pass_xsph = _make_pass("xsph")


def _pad(x):
    return jnp.concatenate([x, jnp.zeros((_NP - _N,), jnp.float32)])


def _combine(o):
    h = o.reshape(_NC, _NP)
    return h[0, :_N] + h[1, :_N]


def kernel(locs, vel, edge_index):
    src2 = edge_index[0].reshape(_E // _SUB, _SUB)
    dst2 = edge_index[1].reshape(_E // _SUB, _SUB)
    zrows = jnp.zeros((_ROWS,), jnp.float32)

    gravity = jnp.array([0.0, -9.8, 0.0], jnp.float32)
    v = vel + _DT * gravity
    speed = jnp.sqrt(jnp.sum(v * v, axis=-1, keepdims=True) + 1e-12)
    v = v * jnp.minimum(1.0, _MAX_VEL / speed)
    pos = locs + _DT * v

    for _ in range(3):
        px, py, pz = (_pad(pos[:, k]) for k in range(3))
        ow, ogx, ogy, ogz, og2, cc, cx, cy, cz = _pass_density(
            px, py, pz, src2, dst2, zrows)
        rho = _combine(ow) + _W0
        C = rho / _REST - 1.0
        gx, gy, gz = _combine(ogx), _combine(ogy), _combine(ogz)
        sum_g2 = _combine(og2) + gx * gx + gy * gy + gz * gz
        lam = -C / (sum_g2 + _EPS)
        odx, ody, odz = _pass_dp(
            _pad(lam), src2, dst2, zrows, cc, cx, cy, cz)
        dp = jnp.stack([_combine(odx), _combine(ody), _combine(odz)], axis=1)
        pos = pos + dp

    new_vel = (pos - locs) / _DT
    px, py, pz = (_pad(pos[:, k]) for k in range(3))
    vx, vy, vz = (_pad(new_vel[:, k]) for k in range(3))
    oxx, oxy, oxz = _pass_xsph(px, py, pz, vx, vy, vz, src2, dst2, zrows)
    xsph = jnp.stack([_combine(oxx), _combine(oxy), _combine(oxz)], axis=1)
    new_vel = new_vel + (0.01 * _VISC * _DT / _REST) * xsph
    return jnp.stack([pos, new_vel])
